# Initial kernel scaffold; baseline (speedup 1.0000x reference)
import jax
import jax.numpy as jnp
from jax import lax
from jax.experimental import pallas as pl
from jax.experimental.pallas import tpu as pltpu
from jax.experimental.pallas import tpu_sc as plsc

N_NODES = 10000
N_EDGES = 320000
D = 128
DH = 64
N_PAD = 10240
CHUNK = 128
NT = 16
ROWS_PER_TILE = N_PAD // NT

DEG_TILES = 32
DEG_CHUNKS = 79
AGG_CHUNKS = 158
E_PAD = DEG_TILES * DEG_CHUNKS * CHUNK

_MESH = plsc.VectorSubcoreMesh(core_axis_name="c", subcore_axis_name="s")


def _deg_body(dst_hbm, zeros_hbm, ones_hbm, deg_out, dst_v, ones_v, deg_sh):
    c = lax.axis_index("c")
    s = lax.axis_index("s")
    w = c * NT + s
    r0 = s * ROWS_PER_TILE
    pltpu.sync_copy(zeros_hbm.at[pl.ds(r0, ROWS_PER_TILE)],
                    deg_sh.at[pl.ds(r0, ROWS_PER_TILE)])
    pltpu.sync_copy(ones_hbm, ones_v)
    pltpu.sync_copy(dst_hbm.at[w], dst_v)
    plsc.subcore_barrier()

    def body(j, carry):
        pltpu.sync_copy(ones_v, deg_sh.at[dst_v.at[j]], add=True)
        return carry

    lax.fori_loop(0, DEG_CHUNKS, body, 0)
    plsc.subcore_barrier()
    pltpu.sync_copy(deg_sh.at[pl.ds(r0, ROWS_PER_TILE)],
                    deg_out.at[c, pl.ds(r0, ROWS_PER_TILE)])


_deg_call = pl.kernel(
    _deg_body,
    out_type=jax.ShapeDtypeStruct((2, N_PAD), jnp.float32),
    mesh=_MESH,
    scratch_types=[
        pltpu.VMEM((DEG_CHUNKS, CHUNK), jnp.int32),
        pltpu.VMEM((CHUNK,), jnp.float32),
        pltpu.VMEM_SHARED((N_PAD,), jnp.float32),
    ],
)


def _agg_body(h0_hbm, h1_hbm, src_hbm, dst_hbm, zrows_hbm, agg_out,
              src_v, dst_v, rows_v, agg_sh):
    c = lax.axis_index("c")
    s = lax.axis_index("s")
    r0 = s * ROWS_PER_TILE
    pltpu.sync_copy(zrows_hbm, agg_sh.at[pl.ds(r0, ROWS_PER_TILE)])
    pltpu.sync_copy(src_hbm.at[s], src_v)
    pltpu.sync_copy(dst_hbm.at[s], dst_v)
    plsc.subcore_barrier()

    def edge_loop(h_hbm):
        def body(j, carry):
            pltpu.sync_copy(h_hbm.at[src_v.at[j]], rows_v)
            pltpu.sync_copy(rows_v, agg_sh.at[dst_v.at[j]], add=True)
            return carry
        lax.fori_loop(0, AGG_CHUNKS, body, 0)

    pl.when(c == 0)(lambda: edge_loop(h0_hbm))
    pl.when(c == 1)(lambda: edge_loop(h1_hbm))
    plsc.subcore_barrier()
    pltpu.sync_copy(agg_sh.at[pl.ds(r0, ROWS_PER_TILE)],
                    agg_out.at[c, pl.ds(r0, ROWS_PER_TILE)])


_agg_call = pl.kernel(
    _agg_body,
    out_type=jax.ShapeDtypeStruct((2, N_PAD, DH), jnp.float32),
    mesh=_MESH,
    scratch_types=[
        pltpu.VMEM((AGG_CHUNKS, CHUNK), jnp.int32),
        pltpu.VMEM((AGG_CHUNKS, CHUNK), jnp.int32),
        pltpu.VMEM((CHUNK, DH), jnp.float32),
        pltpu.VMEM_SHARED((N_PAD, DH), jnp.float32),
    ],
)

BLK = 512


def _dis_from(deg_ref):
    deg = deg_ref[0] + deg_ref[1]
    return jnp.where(deg > 0, lax.rsqrt(jnp.maximum(deg, 1e-12)), 0.0)


def _mm_body(x_ref, w_ref, deg_ref, h0_ref, h1_ref):
    h = jnp.dot(x_ref[...], w_ref[...], preferred_element_type=jnp.float32)
    hp = h * _dis_from(deg_ref)[:, None]
    h0_ref[...] = hp[:, :DH]
    h1_ref[...] = hp[:, DH:]


def _mm_call(x_pad, W, deg2):
    return pl.pallas_call(
        _mm_body,
        grid=(N_PAD // BLK,),
        in_specs=[
            pl.BlockSpec((BLK, D), lambda i: (i, 0)),
            pl.BlockSpec((D, D), lambda i: (0, 0)),
            pl.BlockSpec((2, BLK), lambda i: (0, i)),
        ],
        out_specs=[
            pl.BlockSpec((BLK, DH), lambda i: (i, 0)),
            pl.BlockSpec((BLK, DH), lambda i: (i, 0)),
        ],
        out_shape=[
            jax.ShapeDtypeStruct((N_PAD, DH), jnp.float32),
            jax.ShapeDtypeStruct((N_PAD, DH), jnp.float32),
        ],
    )(x_pad, W, deg2)


def _fin_body(agg_ref, deg_ref, b_ref, o_ref):
    m = jnp.concatenate([agg_ref[0], agg_ref[1]], axis=-1)
    o_ref[...] = jnp.maximum(m * _dis_from(deg_ref)[:, None] + b_ref[...], 0.0)


def _fin_call(agg2, deg2, b2):
    return pl.pallas_call(
        _fin_body,
        grid=(N_PAD // BLK,),
        in_specs=[
            pl.BlockSpec((2, BLK, DH), lambda i: (0, i, 0)),
            pl.BlockSpec((2, BLK), lambda i: (0, i)),
            pl.BlockSpec((1, D), lambda i: (0, 0)),
        ],
        out_specs=pl.BlockSpec((BLK, D), lambda i: (i, 0)),
        out_shape=jax.ShapeDtypeStruct((N_PAD, D), jnp.float32),
    )(agg2, deg2, b2)


def kernel(x, edge_index, W, b):
    src = edge_index[0]
    dst = edge_index[1]
    pad_n = E_PAD - N_EDGES
    pad_idx = N_NODES + (jnp.arange(pad_n, dtype=jnp.int32) % (N_PAD - N_NODES))
    srcp = jnp.concatenate([src, pad_idx])
    dstp = jnp.concatenate([dst, pad_idx])
    dst1 = dstp.reshape(DEG_TILES, DEG_CHUNKS, CHUNK)
    src2 = srcp.reshape(NT, AGG_CHUNKS, CHUNK)
    dst2 = dstp.reshape(NT, AGG_CHUNKS, CHUNK)
    x_pad = jnp.pad(x, ((0, N_PAD - N_NODES), (0, 0)))
    zeros_n = jnp.zeros((N_PAD,), jnp.float32)
    ones_c = jnp.ones((CHUNK,), jnp.float32)
    zeros_rows = jnp.zeros((ROWS_PER_TILE, DH), jnp.float32)

    deg2 = _deg_call(dst1, zeros_n, ones_c)
    h0, h1 = _mm_call(x_pad, W, deg2)
    agg2 = _agg_call(h0, h1, src2, dst2, zeros_rows)
    out = _fin_call(agg2, deg2, b.reshape(1, D))
    return out[:N_NODES]


# speedup vs baseline: 26.9799x; 26.9799x over previous
import jax
import jax.numpy as jnp
from jax import lax
from jax.experimental import pallas as pl
from jax.experimental.pallas import tpu as pltpu
from jax.experimental.pallas import tpu_sc as plsc

N_NODES = 10000
N_EDGES = 320000
D = 128
N_PAD = 10240
CHUNK = 128
NT = 16
ROWS_PER_TILE = N_PAD // NT

TILES = 32
CHUNKS = 79
E_PAD = TILES * CHUNKS * CHUNK

_MESH = plsc.VectorSubcoreMesh(core_axis_name="c", subcore_axis_name="s")


def _deg_body(dst_hbm, zeros_hbm, ones_hbm, deg_out, dst_v, ones_v, deg_sh):
    c = lax.axis_index("c")
    s = lax.axis_index("s")
    w = c * NT + s
    r0 = s * ROWS_PER_TILE
    pltpu.sync_copy(zeros_hbm.at[pl.ds(r0, ROWS_PER_TILE)],
                    deg_sh.at[pl.ds(r0, ROWS_PER_TILE)])
    pltpu.sync_copy(ones_hbm, ones_v)
    pltpu.sync_copy(dst_hbm.at[w], dst_v)
    plsc.subcore_barrier()

    def body(j, carry):
        pltpu.sync_copy(ones_v, deg_sh.at[dst_v.at[j]], add=True)
        return carry

    lax.fori_loop(0, CHUNKS, body, 0)
    plsc.subcore_barrier()
    pltpu.sync_copy(deg_sh.at[pl.ds(r0, ROWS_PER_TILE)],
                    deg_out.at[c, pl.ds(r0, ROWS_PER_TILE)])


_deg_call = pl.kernel(
    _deg_body,
    out_type=jax.ShapeDtypeStruct((2, N_PAD), jnp.float32),
    mesh=_MESH,
    scratch_types=[
        pltpu.VMEM((CHUNKS, CHUNK), jnp.int32),
        pltpu.VMEM((CHUNK,), jnp.float32),
        pltpu.VMEM_SHARED((N_PAD,), jnp.float32),
    ],
)


def _agg_body(hp_hbm, src_hbm, dst_hbm, agg_out, src_v, dst_v, rows_v, agg_sh):
    c = lax.axis_index("c")
    s = lax.axis_index("s")
    w = c * NT + s
    r0 = s * ROWS_PER_TILE

    def zbody(i, carry):
        for j in range(D // 16):
            rows_v[i, pl.ds(j * 16, 16)] = jnp.zeros((16,), jnp.float32)
        return carry

    lax.fori_loop(0, CHUNK, zbody, 0)
    for k in range(ROWS_PER_TILE // CHUNK):
        pltpu.sync_copy(rows_v, agg_sh.at[pl.ds(r0 + k * CHUNK, CHUNK)])
    pltpu.sync_copy(src_hbm.at[w], src_v)
    pltpu.sync_copy(dst_hbm.at[w], dst_v)
    plsc.subcore_barrier()

    def body(j, carry):
        pltpu.sync_copy(hp_hbm.at[src_v.at[j]], rows_v)
        pltpu.sync_copy(rows_v, agg_sh.at[dst_v.at[j]], add=True)
        return carry

    lax.fori_loop(0, CHUNKS, body, 0)
    plsc.subcore_barrier()
    pltpu.sync_copy(agg_sh.at[pl.ds(r0, ROWS_PER_TILE)],
                    agg_out.at[c, pl.ds(r0, ROWS_PER_TILE)])


_agg_call = pl.kernel(
    _agg_body,
    out_type=jax.ShapeDtypeStruct((2, N_PAD, D), jnp.float32),
    mesh=_MESH,
    scratch_types=[
        pltpu.VMEM((CHUNKS, CHUNK), jnp.int32),
        pltpu.VMEM((CHUNKS, CHUNK), jnp.int32),
        pltpu.VMEM((CHUNK, D), jnp.float32),
        pltpu.VMEM_SHARED((N_PAD, D), jnp.float32),
    ],
)

BLK = 512


def _dis_from(deg_ref):
    deg = deg_ref[0] + deg_ref[1]
    return jnp.where(deg > 0, lax.rsqrt(jnp.maximum(deg, 1e-12)), 0.0)


def _mm_body(x_ref, w_ref, deg_ref, hp_ref):
    h = jnp.dot(x_ref[...], w_ref[...], preferred_element_type=jnp.float32)
    hp_ref[...] = h * _dis_from(deg_ref)[:, None]


def _mm_call(x_pad, W, deg2):
    return pl.pallas_call(
        _mm_body,
        grid=(N_PAD // BLK,),
        in_specs=[
            pl.BlockSpec((BLK, D), lambda i: (i, 0)),
            pl.BlockSpec((D, D), lambda i: (0, 0)),
            pl.BlockSpec((2, BLK), lambda i: (0, i)),
        ],
        out_specs=pl.BlockSpec((BLK, D), lambda i: (i, 0)),
        out_shape=jax.ShapeDtypeStruct((N_PAD, D), jnp.float32),
    )(x_pad, W, deg2)


def _fin_body(agg_ref, deg_ref, b_ref, o_ref):
    m = agg_ref[0] + agg_ref[1]
    o_ref[...] = jnp.maximum(m * _dis_from(deg_ref)[:, None] + b_ref[...], 0.0)


def _fin_call(agg2, deg2, b2):
    return pl.pallas_call(
        _fin_body,
        grid=(N_PAD // BLK,),
        in_specs=[
            pl.BlockSpec((2, BLK, D), lambda i: (0, i, 0)),
            pl.BlockSpec((2, BLK), lambda i: (0, i)),
            pl.BlockSpec((1, D), lambda i: (0, 0)),
        ],
        out_specs=pl.BlockSpec((BLK, D), lambda i: (i, 0)),
        out_shape=jax.ShapeDtypeStruct((N_PAD, D), jnp.float32),
    )(agg2, deg2, b2)


def kernel(x, edge_index, W, b):
    src = edge_index[0]
    dst = edge_index[1]
    pad_n = E_PAD - N_EDGES
    pad_idx = N_NODES + (jnp.arange(pad_n, dtype=jnp.int32) % (N_PAD - N_NODES))
    srcp = jnp.concatenate([src, pad_idx]).reshape(TILES, CHUNKS, CHUNK)
    dstp = jnp.concatenate([dst, pad_idx]).reshape(TILES, CHUNKS, CHUNK)
    x_pad = jnp.pad(x, ((0, N_PAD - N_NODES), (0, 0)))
    zeros_n = jnp.zeros((N_PAD,), jnp.float32)
    ones_c = jnp.ones((CHUNK,), jnp.float32)

    deg2 = _deg_call(dstp, zeros_n, ones_c)
    hp = _mm_call(x_pad, W, deg2)
    agg2 = _agg_call(hp, srcp, dstp)
    out = _fin_call(agg2, deg2, b.reshape(1, D))
    return out[:N_NODES]


# speedup vs baseline: 35.1587x; 1.3031x over previous
import jax
import jax.numpy as jnp
from jax import lax
from jax.experimental import pallas as pl
from jax.experimental.pallas import tpu as pltpu
from jax.experimental.pallas import tpu_sc as plsc

N_NODES = 10000
N_EDGES = 320000
D = 128
DH = 64
N_PAD = 10112
CHUNK = 128
NT = 16
ROWS_PER_TILE = N_PAD // NT

E_PAD = 327680
DEG_CHUNKS = 80
AGG_CHUNKS = 160
NBUF = 4

_MESH = plsc.VectorSubcoreMesh(core_axis_name="c", subcore_axis_name="s")


def _tile_1d_ranges(s, fn):
    @pl.when(s < NT - 1)
    def _():
        fn(s * 640, 640)

    @pl.when(s == NT - 1)
    def _():
        fn((NT - 1) * 640, N_PAD - (NT - 1) * 640)


def _deg_body(dst_hbm, zeros_hbm, ones_hbm, deg_out_a, deg_out_b,
              dst_v, ones_v, deg_sh, deg_sem):
    c = lax.axis_index("c")
    s = lax.axis_index("s")
    w = c * NT + s
    _tile_1d_ranges(s, lambda lo, n: pltpu.sync_copy(
        zeros_hbm.at[pl.ds(lo, n)], deg_sh.at[pl.ds(lo, n)]))
    pltpu.sync_copy(ones_hbm, ones_v)
    pltpu.sync_copy(dst_hbm.at[w], dst_v)
    plsc.subcore_barrier()

    def body(j, carry):
        pltpu.async_copy(ones_v, deg_sh.at[dst_v.at[j]], deg_sem, add=True)
        return carry

    lax.fori_loop(0, DEG_CHUNKS, body, 0)

    def drain(j, carry):
        pltpu.make_async_copy(ones_v, deg_sh.at[dst_v.at[j]], deg_sem).wait()
        return carry

    lax.fori_loop(0, DEG_CHUNKS, drain, 0)
    plsc.subcore_barrier()

    @pl.when(c == 0)
    def _():
        _tile_1d_ranges(s, lambda lo, n: pltpu.sync_copy(
            deg_sh.at[pl.ds(lo, n)], deg_out_a.at[pl.ds(lo, n)]))

    @pl.when(c == 1)
    def _():
        _tile_1d_ranges(s, lambda lo, n: pltpu.sync_copy(
            deg_sh.at[pl.ds(lo, n)], deg_out_b.at[pl.ds(lo, n)]))


_deg_call = pl.kernel(
    _deg_body,
    out_type=(jax.ShapeDtypeStruct((N_PAD,), jnp.float32),
              jax.ShapeDtypeStruct((N_PAD,), jnp.float32)),
    mesh=_MESH,
    scratch_types=[
        pltpu.VMEM((DEG_CHUNKS, CHUNK), jnp.int32),
        pltpu.VMEM((CHUNK,), jnp.float32),
        pltpu.VMEM_SHARED((N_PAD,), jnp.float32),
        pltpu.SemaphoreType.DMA,
    ],
)


def _agg_body(h0_hbm, h1_hbm, src_hbm, dst_hbm, agg_out_a, agg_out_b,
              src_v, dst_v, rows0, rows1, rows2, rows3, agg_sh, *sems):
    rows = (rows0, rows1, rows2, rows3)
    gsem = sems[:NBUF]
    ssem = sems[NBUF:]
    c = lax.axis_index("c")
    s = lax.axis_index("s")

    def zbody(i, carry):
        for j in range(DH // 16):
            rows0[i, pl.ds(j * 16, 16)] = jnp.zeros((16,), jnp.float32)
        return carry

    lax.fori_loop(0, CHUNK, zbody, 0)
    r0 = s * ROWS_PER_TILE
    nfull = ROWS_PER_TILE // CHUNK
    for k in range(nfull):
        pltpu.sync_copy(rows0, agg_sh.at[pl.ds(r0 + k * CHUNK, CHUNK)])
    rem = ROWS_PER_TILE % CHUNK
    if rem:
        pltpu.sync_copy(rows0.at[pl.ds(0, rem)],
                        agg_sh.at[pl.ds(r0 + nfull * CHUNK, rem)])
    pltpu.sync_copy(src_hbm.at[s], src_v)
    pltpu.sync_copy(dst_hbm.at[s], dst_v)
    plsc.subcore_barrier()

    def edge_loop(h_hbm):
        def start_g(t, b):
            pltpu.async_copy(h_hbm.at[src_v.at[t]], rows[b], gsem[b])

        def wait_g(t, b):
            pltpu.make_async_copy(h_hbm.at[src_v.at[t]], rows[b],
                                  gsem[b]).wait()

        def start_s(t, b):
            pltpu.async_copy(rows[b], agg_sh.at[dst_v.at[t]], ssem[b],
                             add=True)

        def wait_s(t, b):
            pltpu.make_async_copy(rows[b], agg_sh.at[dst_v.at[t]],
                                  ssem[b]).wait()

        def slot(t, b, with_ws, with_sg):
            wait_g(t, b)
            start_s(t, b)
            if with_ws:
                wait_s(t - 2, (b + 2) % NBUF)
            if with_sg:
                start_g(t + 2, (b + 2) % NBUF)

        start_g(0, 0)
        start_g(1, 1)
        slot(0, 0, False, True)
        slot(1, 1, False, True)
        slot(2, 2, True, True)
        slot(3, 3, True, True)

        def round_body(g, carry):
            t0 = g * NBUF
            for b in range(NBUF):
                slot(t0 + b, b, True, True)
            return carry

        lax.fori_loop(1, AGG_CHUNKS // NBUF - 1, round_body, 0)
        t0 = AGG_CHUNKS - NBUF
        slot(t0 + 0, 0, True, True)
        slot(t0 + 1, 1, True, True)
        slot(t0 + 2, 2, True, False)
        slot(t0 + 3, 3, True, False)
        wait_s(AGG_CHUNKS - 2, (AGG_CHUNKS - 2) % NBUF)
        wait_s(AGG_CHUNKS - 1, (AGG_CHUNKS - 1) % NBUF)

    @pl.when(c == 0)
    def _():
        edge_loop(h0_hbm)

    @pl.when(c == 1)
    def _():
        edge_loop(h1_hbm)

    plsc.subcore_barrier()

    @pl.when(c == 0)
    def _():
        pltpu.sync_copy(agg_sh.at[pl.ds(r0, ROWS_PER_TILE)],
                        agg_out_a.at[pl.ds(r0, ROWS_PER_TILE)])

    @pl.when(c == 1)
    def _():
        pltpu.sync_copy(agg_sh.at[pl.ds(r0, ROWS_PER_TILE)],
                        agg_out_b.at[pl.ds(r0, ROWS_PER_TILE)])


_agg_call = pl.kernel(
    _agg_body,
    out_type=(jax.ShapeDtypeStruct((N_PAD, DH), jnp.float32),
              jax.ShapeDtypeStruct((N_PAD, DH), jnp.float32)),
    mesh=_MESH,
    scratch_types=[
        pltpu.VMEM((AGG_CHUNKS, CHUNK), jnp.int32),
        pltpu.VMEM((AGG_CHUNKS, CHUNK), jnp.int32),
        pltpu.VMEM((CHUNK, DH), jnp.float32),
        pltpu.VMEM((CHUNK, DH), jnp.float32),
        pltpu.VMEM((CHUNK, DH), jnp.float32),
        pltpu.VMEM((CHUNK, DH), jnp.float32),
        pltpu.VMEM_SHARED((N_PAD, DH), jnp.float32),
    ] + [pltpu.SemaphoreType.DMA] * (2 * NBUF),
    compiler_params=pltpu.CompilerParams(use_tc_tiling_on_sc=False),
)

BLK = N_PAD


def _dis_from(deg_ref):
    deg = deg_ref[0] + deg_ref[1]
    return jnp.where(deg > 0, lax.rsqrt(jnp.maximum(deg, 1e-12)), 0.0)


def _mm_body(x_ref, w_ref, deg_ref, h0_ref, h1_ref):
    h = jnp.dot(x_ref[...], w_ref[...], preferred_element_type=jnp.float32)
    hp = h * _dis_from(deg_ref)[:, None]
    h0_ref[...] = hp[:, :DH]
    h1_ref[...] = hp[:, DH:]


def _mm_call(x_pad, W, deg2):
    return pl.pallas_call(
        _mm_body,
        grid=(N_PAD // BLK,),
        in_specs=[
            pl.BlockSpec((BLK, D), lambda i: (i, 0)),
            pl.BlockSpec((D, D), lambda i: (0, 0)),
            pl.BlockSpec((2, BLK), lambda i: (0, i)),
        ],
        out_specs=[
            pl.BlockSpec((BLK, DH), lambda i: (i, 0)),
            pl.BlockSpec((BLK, DH), lambda i: (i, 0)),
        ],
        out_shape=[
            jax.ShapeDtypeStruct((N_PAD, DH), jnp.float32),
            jax.ShapeDtypeStruct((N_PAD, DH), jnp.float32),
        ],
    )(x_pad, W, deg2)


def _fin_body(agg_a_ref, agg_b_ref, deg_ref, b_ref, o_ref):
    m = jnp.concatenate([agg_a_ref[...], agg_b_ref[...]], axis=-1)
    o_ref[...] = jnp.maximum(m * _dis_from(deg_ref)[:, None] + b_ref[...], 0.0)


def _fin_call(agg_a, agg_b, deg2, b2):
    return pl.pallas_call(
        _fin_body,
        grid=(N_PAD // BLK,),
        in_specs=[
            pl.BlockSpec((BLK, DH), lambda i: (i, 0)),
            pl.BlockSpec((BLK, DH), lambda i: (i, 0)),
            pl.BlockSpec((2, BLK), lambda i: (0, i)),
            pl.BlockSpec((1, D), lambda i: (0, 0)),
        ],
        out_specs=pl.BlockSpec((BLK, D), lambda i: (i, 0)),
        out_shape=jax.ShapeDtypeStruct((N_PAD, D), jnp.float32),
    )(agg_a, agg_b, deg2, b2)


def kernel(x, edge_index, W, b):
    src = edge_index[0]
    dst = edge_index[1]
    pad_n = E_PAD - N_EDGES
    pad_idx = N_NODES + (jnp.arange(pad_n, dtype=jnp.int32) % (N_PAD - N_NODES))
    srcp = jnp.concatenate([src, pad_idx])
    dstp = jnp.concatenate([dst, pad_idx])
    dst_deg = dstp.reshape(32, DEG_CHUNKS, CHUNK)
    src_agg = srcp.reshape(NT, AGG_CHUNKS, CHUNK)
    dst_agg = dstp.reshape(NT, AGG_CHUNKS, CHUNK)
    x_pad = jnp.pad(x, ((0, N_PAD - N_NODES), (0, 0)))
    zeros_n = jnp.zeros((N_PAD,), jnp.float32)
    ones_c = jnp.ones((CHUNK,), jnp.float32)

    deg_a, deg_b = _deg_call(dst_deg, zeros_n, ones_c)
    deg2 = jnp.stack([deg_a, deg_b])
    h0, h1 = _mm_call(x_pad, W, deg2)
    agg_a, agg_b = _agg_call(h0, h1, src_agg, dst_agg)
    out = _fin_call(agg_a, agg_b, deg2, b.reshape(1, D))
    return out[:N_NODES]


# speedup vs baseline: 39.0908x; 1.1118x over previous
import jax
import jax.numpy as jnp
from jax import lax
from jax.experimental import pallas as pl
from jax.experimental.pallas import tpu as pltpu
from jax.experimental.pallas import tpu_sc as plsc

N_NODES = 10000
N_EDGES = 320000
D = 128
DH = 64
N_PAD = 10112
CHUNK = 128
NT = 16
ROWS_PER_TILE = N_PAD // NT

E_PAD = 327680
DEG_CHUNKS = 80
AGG_CHUNKS = 160
NBUF = 5

_MESH = plsc.VectorSubcoreMesh(core_axis_name="c", subcore_axis_name="s")


def _tile_1d_ranges(s, fn):
    @pl.when(s < NT - 1)
    def _():
        fn(s * 640, 640)

    @pl.when(s == NT - 1)
    def _():
        fn((NT - 1) * 640, N_PAD - (NT - 1) * 640)


def _deg_body(dst_hbm, zeros_hbm, ones_hbm, deg_out_a, deg_out_b,
              dst_v, ones_v, deg_sh, deg_sem):
    c = lax.axis_index("c")
    s = lax.axis_index("s")
    w = c * NT + s
    _tile_1d_ranges(s, lambda lo, n: pltpu.sync_copy(
        zeros_hbm.at[pl.ds(lo, n)], deg_sh.at[pl.ds(lo, n)]))
    pltpu.sync_copy(ones_hbm, ones_v)
    pltpu.sync_copy(dst_hbm.at[w], dst_v)
    plsc.subcore_barrier()

    def body(j, carry):
        pltpu.async_copy(ones_v, deg_sh.at[dst_v.at[j]], deg_sem, add=True)
        return carry

    lax.fori_loop(0, DEG_CHUNKS, body, 0)

    def drain(j, carry):
        pltpu.make_async_copy(ones_v, deg_sh.at[dst_v.at[j]], deg_sem).wait()
        return carry

    lax.fori_loop(0, DEG_CHUNKS, drain, 0)
    plsc.subcore_barrier()

    @pl.when(c == 0)
    def _():
        _tile_1d_ranges(s, lambda lo, n: pltpu.sync_copy(
            deg_sh.at[pl.ds(lo, n)], deg_out_a.at[pl.ds(lo, n)]))

    @pl.when(c == 1)
    def _():
        _tile_1d_ranges(s, lambda lo, n: pltpu.sync_copy(
            deg_sh.at[pl.ds(lo, n)], deg_out_b.at[pl.ds(lo, n)]))


_deg_call = pl.kernel(
    _deg_body,
    out_type=(jax.ShapeDtypeStruct((N_PAD,), jnp.float32),
              jax.ShapeDtypeStruct((N_PAD,), jnp.float32)),
    mesh=_MESH,
    scratch_types=[
        pltpu.VMEM((DEG_CHUNKS, CHUNK), jnp.int32),
        pltpu.VMEM((CHUNK,), jnp.float32),
        pltpu.VMEM_SHARED((N_PAD,), jnp.float32),
        pltpu.SemaphoreType.DMA,
    ],
)


def _agg_body(h0_hbm, h1_hbm, src_hbm, dst_hbm, agg_out_a, agg_out_b,
              src_v, dst_v, rows0, rows1, rows2, rows3, rows4, agg_sh, *sems):
    rows = (rows0, rows1, rows2, rows3, rows4)
    gsem = sems[:NBUF]
    ssem = sems[NBUF:]
    c = lax.axis_index("c")
    s = lax.axis_index("s")

    def zbody(i, carry):
        for j in range(DH // 16):
            rows0[i, pl.ds(j * 16, 16)] = jnp.zeros((16,), jnp.float32)
        return carry

    lax.fori_loop(0, CHUNK, zbody, 0)
    r0 = s * ROWS_PER_TILE
    nfull = ROWS_PER_TILE // CHUNK
    for k in range(nfull):
        pltpu.sync_copy(rows0, agg_sh.at[pl.ds(r0 + k * CHUNK, CHUNK)])
    rem = ROWS_PER_TILE % CHUNK
    if rem:
        pltpu.sync_copy(rows0.at[pl.ds(0, rem)],
                        agg_sh.at[pl.ds(r0 + nfull * CHUNK, rem)])
    pltpu.sync_copy(src_hbm.at[s], src_v)
    pltpu.sync_copy(dst_hbm.at[s], dst_v)
    plsc.subcore_barrier()

    def edge_loop(h_hbm):
        def start_g(t, b):
            pltpu.async_copy(h_hbm.at[src_v.at[t]], rows[b], gsem[b])

        def wait_g(t, b):
            pltpu.make_async_copy(h_hbm.at[src_v.at[t]], rows[b],
                                  gsem[b]).wait()

        def start_s(t, b):
            pltpu.async_copy(rows[b], agg_sh.at[dst_v.at[t]], ssem[b],
                             add=True)

        def wait_s(t, b):
            pltpu.make_async_copy(rows[b], agg_sh.at[dst_v.at[t]],
                                  ssem[b]).wait()

        def slot(t, b, with_ws, with_sg):
            wait_g(t, b)
            start_s(t, b)
            if with_ws:
                wait_s(t - 2, (b + 3) % NBUF)
            if with_sg:
                start_g(t + 3, (b + 3) % NBUF)

        start_g(0, 0)
        start_g(1, 1)
        start_g(2, 2)
        slot(0, 0, False, True)
        slot(1, 1, False, True)
        slot(2, 2, True, True)
        slot(3, 3, True, True)
        slot(4, 4, True, True)

        def round_body(g, carry):
            t0 = g * NBUF
            for b in range(NBUF):
                slot(t0 + b, b, True, True)
            return carry

        lax.fori_loop(1, AGG_CHUNKS // NBUF - 1, round_body, 0)
        t0 = AGG_CHUNKS - NBUF
        slot(t0 + 0, 0, True, True)
        slot(t0 + 1, 1, True, True)
        slot(t0 + 2, 2, True, False)
        slot(t0 + 3, 3, True, False)
        slot(t0 + 4, 4, True, False)
        wait_s(AGG_CHUNKS - 2, (AGG_CHUNKS - 2) % NBUF)
        wait_s(AGG_CHUNKS - 1, (AGG_CHUNKS - 1) % NBUF)

    @pl.when(c == 0)
    def _():
        edge_loop(h0_hbm)

    @pl.when(c == 1)
    def _():
        edge_loop(h1_hbm)

    plsc.subcore_barrier()

    @pl.when(c == 0)
    def _():
        pltpu.sync_copy(agg_sh.at[pl.ds(r0, ROWS_PER_TILE)],
                        agg_out_a.at[pl.ds(r0, ROWS_PER_TILE)])

    @pl.when(c == 1)
    def _():
        pltpu.sync_copy(agg_sh.at[pl.ds(r0, ROWS_PER_TILE)],
                        agg_out_b.at[pl.ds(r0, ROWS_PER_TILE)])


_agg_call = pl.kernel(
    _agg_body,
    out_type=(jax.ShapeDtypeStruct((N_PAD, DH), jnp.float32),
              jax.ShapeDtypeStruct((N_PAD, DH), jnp.float32)),
    mesh=_MESH,
    scratch_types=[
        pltpu.VMEM((AGG_CHUNKS, CHUNK), jnp.int32),
        pltpu.VMEM((AGG_CHUNKS, CHUNK), jnp.int32),
        pltpu.VMEM((CHUNK, DH), jnp.float32),
        pltpu.VMEM((CHUNK, DH), jnp.float32),
        pltpu.VMEM((CHUNK, DH), jnp.float32),
        pltpu.VMEM((CHUNK, DH), jnp.float32),
        pltpu.VMEM((CHUNK, DH), jnp.float32),
        pltpu.VMEM_SHARED((N_PAD, DH), jnp.float32),
    ] + [pltpu.SemaphoreType.DMA] * (2 * NBUF),
    compiler_params=pltpu.CompilerParams(use_tc_tiling_on_sc=False),
)

BLK = N_PAD


def _dis_from(deg_ref):
    deg = deg_ref[0] + deg_ref[1]
    return jnp.where(deg > 0, lax.rsqrt(jnp.maximum(deg, 1e-12)), 0.0)


def _mm_body(x_ref, w_ref, deg_ref, h0_ref, h1_ref):
    h = jnp.dot(x_ref[...], w_ref[...], preferred_element_type=jnp.float32)
    hp = h * _dis_from(deg_ref)[:, None]
    h0_ref[...] = hp[:, :DH]
    h1_ref[...] = hp[:, DH:]


def _mm_call(x_pad, W, deg2):
    return pl.pallas_call(
        _mm_body,
        grid=(N_PAD // BLK,),
        in_specs=[
            pl.BlockSpec((BLK, D), lambda i: (i, 0)),
            pl.BlockSpec((D, D), lambda i: (0, 0)),
            pl.BlockSpec((2, BLK), lambda i: (0, i)),
        ],
        out_specs=[
            pl.BlockSpec((BLK, DH), lambda i: (i, 0)),
            pl.BlockSpec((BLK, DH), lambda i: (i, 0)),
        ],
        out_shape=[
            jax.ShapeDtypeStruct((N_PAD, DH), jnp.float32),
            jax.ShapeDtypeStruct((N_PAD, DH), jnp.float32),
        ],
    )(x_pad, W, deg2)


def _fin_body(agg_a_ref, agg_b_ref, deg_ref, b_ref, o_ref):
    m = jnp.concatenate([agg_a_ref[...], agg_b_ref[...]], axis=-1)
    o_ref[...] = jnp.maximum(m * _dis_from(deg_ref)[:, None] + b_ref[...], 0.0)


def _fin_call(agg_a, agg_b, deg2, b2):
    return pl.pallas_call(
        _fin_body,
        grid=(N_PAD // BLK,),
        in_specs=[
            pl.BlockSpec((BLK, DH), lambda i: (i, 0)),
            pl.BlockSpec((BLK, DH), lambda i: (i, 0)),
            pl.BlockSpec((2, BLK), lambda i: (0, i)),
            pl.BlockSpec((1, D), lambda i: (0, 0)),
        ],
        out_specs=pl.BlockSpec((BLK, D), lambda i: (i, 0)),
        out_shape=jax.ShapeDtypeStruct((N_PAD, D), jnp.float32),
    )(agg_a, agg_b, deg2, b2)


def kernel(x, edge_index, W, b):
    src = edge_index[0]
    dst = edge_index[1]
    pad_n = E_PAD - N_EDGES
    pad_idx = N_NODES + (jnp.arange(pad_n, dtype=jnp.int32) % (N_PAD - N_NODES))
    srcp = jnp.concatenate([src, pad_idx])
    dstp = jnp.concatenate([dst, pad_idx])
    dst_deg = dstp.reshape(32, DEG_CHUNKS, CHUNK)
    src_agg = srcp.reshape(NT, AGG_CHUNKS, CHUNK)
    dst_agg = dstp.reshape(NT, AGG_CHUNKS, CHUNK)
    x_pad = jnp.pad(x, ((0, N_PAD - N_NODES), (0, 0)))
    zeros_n = jnp.zeros((N_PAD,), jnp.float32)
    ones_c = jnp.ones((CHUNK,), jnp.float32)

    deg_a, deg_b = _deg_call(dst_deg, zeros_n, ones_c)
    deg2 = jnp.stack([deg_a, deg_b])
    h0, h1 = _mm_call(x_pad, W, deg2)
    agg_a, agg_b = _agg_call(h0, h1, src_agg, dst_agg)
    out = _fin_call(agg_a, agg_b, deg2, b.reshape(1, D))
    return out[:N_NODES]


# speedup vs baseline: 41.8816x; 1.0714x over previous
import jax
import jax.numpy as jnp
from jax import lax
from jax.experimental import pallas as pl
from jax.experimental.pallas import tpu as pltpu
from jax.experimental.pallas import tpu_sc as plsc

N_NODES = 10000
N_EDGES = 320000
D = 128
DH = 64
N_PAD = 10112
CHUNK = 128
NT = 16
ROWS_PER_TILE = N_PAD // NT

E_PAD = 327680
DEG_CHUNKS = 80
AGG_CHUNKS = 160
NBUF = 5

_MESH = plsc.VectorSubcoreMesh(core_axis_name="c", subcore_axis_name="s")


def _tile_1d_ranges(s, fn):
    @pl.when(s < NT - 1)
    def _():
        fn(s * 640, 640)

    @pl.when(s == NT - 1)
    def _():
        fn((NT - 1) * 640, N_PAD - (NT - 1) * 640)


def _deg_body(dst_hbm, zeros_hbm, ones_hbm, deg_out_a, deg_out_b,
              dst_v, ones_v, deg_sh, deg_sem):
    c = lax.axis_index("c")
    s = lax.axis_index("s")
    w = c * NT + s
    _tile_1d_ranges(s, lambda lo, n: pltpu.sync_copy(
        zeros_hbm.at[pl.ds(lo, n)], deg_sh.at[pl.ds(lo, n)]))
    pltpu.sync_copy(ones_hbm, ones_v)
    pltpu.sync_copy(dst_hbm.at[w], dst_v)
    plsc.subcore_barrier()

    def body(j, carry):
        pltpu.async_copy(ones_v, deg_sh.at[dst_v.at[j]], deg_sem, add=True)
        return carry

    lax.fori_loop(0, DEG_CHUNKS, body, 0)

    def drain(j, carry):
        pltpu.make_async_copy(ones_v, deg_sh.at[dst_v.at[j]], deg_sem).wait()
        return carry

    lax.fori_loop(0, DEG_CHUNKS, drain, 0)
    plsc.subcore_barrier()

    @pl.when(c == 0)
    def _():
        _tile_1d_ranges(s, lambda lo, n: pltpu.sync_copy(
            deg_sh.at[pl.ds(lo, n)], deg_out_a.at[pl.ds(lo, n)]))

    @pl.when(c == 1)
    def _():
        _tile_1d_ranges(s, lambda lo, n: pltpu.sync_copy(
            deg_sh.at[pl.ds(lo, n)], deg_out_b.at[pl.ds(lo, n)]))


_deg_call = pl.kernel(
    _deg_body,
    out_type=(jax.ShapeDtypeStruct((N_PAD,), jnp.float32),
              jax.ShapeDtypeStruct((N_PAD,), jnp.float32)),
    mesh=_MESH,
    scratch_types=[
        pltpu.VMEM((DEG_CHUNKS, CHUNK), jnp.int32),
        pltpu.VMEM((CHUNK,), jnp.float32),
        pltpu.VMEM_SHARED((N_PAD,), jnp.float32),
        pltpu.SemaphoreType.DMA,
    ],
)


def _agg_body(h0_hbm, h1_hbm, src_hbm, dst_hbm, deg_a_hbm, deg_b_hbm, b_hbm,
              out_hbm, src_v, dst_v, rows0, rows1, rows2, rows3, rows4,
              deg_va, deg_vb, dis_v, b_v, agg_sh, *sems):
    rows = (rows0, rows1, rows2, rows3, rows4)
    gsem = sems[:NBUF]
    ssem = sems[NBUF:]
    c = lax.axis_index("c")
    s = lax.axis_index("s")

    def zbody(i, carry):
        for j in range(DH // 16):
            rows0[i, pl.ds(j * 16, 16)] = jnp.zeros((16,), jnp.float32)
        return carry

    lax.fori_loop(0, CHUNK, zbody, 0)
    r0 = s * ROWS_PER_TILE
    nfull = ROWS_PER_TILE // CHUNK
    for k in range(nfull):
        pltpu.sync_copy(rows0, agg_sh.at[pl.ds(r0 + k * CHUNK, CHUNK)])
    rem = ROWS_PER_TILE % CHUNK
    if rem:
        pltpu.sync_copy(rows0.at[pl.ds(0, rem)],
                        agg_sh.at[pl.ds(r0 + nfull * CHUNK, rem)])
    pltpu.sync_copy(src_hbm.at[s], src_v)
    pltpu.sync_copy(dst_hbm.at[s], dst_v)
    plsc.subcore_barrier()

    def edge_loop(h_hbm):
        def start_g(t, b):
            pltpu.async_copy(h_hbm.at[src_v.at[t]], rows[b], gsem[b])

        def wait_g(t, b):
            pltpu.make_async_copy(h_hbm.at[src_v.at[t]], rows[b],
                                  gsem[b]).wait()

        def start_s(t, b):
            pltpu.async_copy(rows[b], agg_sh.at[dst_v.at[t]], ssem[b],
                             add=True)

        def wait_s(t, b):
            pltpu.make_async_copy(rows[b], agg_sh.at[dst_v.at[t]],
                                  ssem[b]).wait()

        def slot(t, b, with_ws, with_sg):
            wait_g(t, b)
            start_s(t, b)
            if with_ws:
                wait_s(t - 2, (b + 3) % NBUF)
            if with_sg:
                start_g(t + 3, (b + 3) % NBUF)

        start_g(0, 0)
        start_g(1, 1)
        start_g(2, 2)
        slot(0, 0, False, True)
        slot(1, 1, False, True)
        slot(2, 2, True, True)
        slot(3, 3, True, True)
        slot(4, 4, True, True)

        def round_body(g, carry):
            t0 = g * NBUF
            for b in range(NBUF):
                slot(t0 + b, b, True, True)
            return carry

        lax.fori_loop(1, AGG_CHUNKS // NBUF - 1, round_body, 0)
        t0 = AGG_CHUNKS - NBUF
        slot(t0 + 0, 0, True, True)
        slot(t0 + 1, 1, True, True)
        slot(t0 + 2, 2, True, False)
        slot(t0 + 3, 3, True, False)
        slot(t0 + 4, 4, True, False)
        wait_s(AGG_CHUNKS - 2, (AGG_CHUNKS - 2) % NBUF)
        wait_s(AGG_CHUNKS - 1, (AGG_CHUNKS - 1) % NBUF)

    @pl.when(c == 0)
    def _():
        edge_loop(h0_hbm)

    @pl.when(c == 1)
    def _():
        edge_loop(h1_hbm)

    plsc.subcore_barrier()

    off = jnp.where(s == NT - 1, 8, 0)
    lo = r0 - off
    pltpu.sync_copy(deg_a_hbm.at[pl.ds(lo, 640)], deg_va)
    pltpu.sync_copy(deg_b_hbm.at[pl.ds(lo, 640)], deg_vb)
    pltpu.sync_copy(b_hbm, b_v)

    def newton(k, carry):
        da = deg_va[pl.ds(k * 16, 16)] + deg_vb[pl.ds(k * 16, 16)]
        xi = plsc.bitcast(da, jnp.int32)
        yi = jnp.int32(0x5F3759DF) - lax.shift_right_logical(xi, 1)
        y = plsc.bitcast(yi, jnp.float32)
        for _ in range(4):
            y = y * (1.5 - 0.5 * da * y * y)
        dis_v[pl.ds(k * 16, 16)] = jnp.where(da > 0, y, 0.0)
        return carry

    lax.fori_loop(0, 640 // 16, newton, 0)
    bvs = [b_v[pl.ds(c * DH + j * 16, 16)] for j in range(DH // 16)]

    FB = 79

    def wr(start, rows_n):
        @pl.when(c == 0)
        def _():
            pltpu.sync_copy(rows0.at[pl.ds(0, rows_n)],
                            out_hbm.at[pl.ds(start, rows_n), pl.ds(0, DH)])

        @pl.when(c == 1)
        def _():
            pltpu.sync_copy(rows0.at[pl.ds(0, rows_n)],
                            out_hbm.at[pl.ds(start, rows_n), pl.ds(DH, DH)])

    for blk in range(ROWS_PER_TILE // FB):
        row_off = blk * FB
        start = r0 + row_off
        pltpu.sync_copy(agg_sh.at[pl.ds(start, FB)], rows0.at[pl.ds(0, FB)])

        def rowfix(r, carry):
            db = plsc.load_gather(
                dis_v, [jnp.full((16,), off + row_off, jnp.int32) + r])
            for j in range(DH // 16):
                v = rows0[r, pl.ds(j * 16, 16)]
                rows0[r, pl.ds(j * 16, 16)] = jnp.maximum(v * db + bvs[j], 0.0)
            return carry

        lax.fori_loop(0, FB, rowfix, 0)

        @pl.when(start + FB <= N_NODES)
        def _():
            wr(start, FB)

        @pl.when(jnp.logical_and(start < N_NODES, start + FB > N_NODES))
        def _():
            wr(start, N_NODES - (15 * ROWS_PER_TILE + 6 * FB))


_agg_call = pl.kernel(
    _agg_body,
    out_type=jax.ShapeDtypeStruct((N_NODES, D), jnp.float32),
    mesh=_MESH,
    scratch_types=[
        pltpu.VMEM((AGG_CHUNKS, CHUNK), jnp.int32),
        pltpu.VMEM((AGG_CHUNKS, CHUNK), jnp.int32),
        pltpu.VMEM((CHUNK, DH), jnp.float32),
        pltpu.VMEM((CHUNK, DH), jnp.float32),
        pltpu.VMEM((CHUNK, DH), jnp.float32),
        pltpu.VMEM((CHUNK, DH), jnp.float32),
        pltpu.VMEM((CHUNK, DH), jnp.float32),
        pltpu.VMEM((640,), jnp.float32),
        pltpu.VMEM((640,), jnp.float32),
        pltpu.VMEM((640,), jnp.float32),
        pltpu.VMEM((D,), jnp.float32),
        pltpu.VMEM_SHARED((N_PAD, DH), jnp.float32),
    ] + [pltpu.SemaphoreType.DMA] * (2 * NBUF),
    compiler_params=pltpu.CompilerParams(use_tc_tiling_on_sc=False,
                                         needs_layout_passes=False),
)

BLK = N_PAD


def _dis_from(deg_ref):
    deg = deg_ref[0] + deg_ref[1]
    return jnp.where(deg > 0, lax.rsqrt(jnp.maximum(deg, 1e-12)), 0.0)


def _mm_body(x_ref, w_ref, deg_ref, h0_ref, h1_ref):
    h = jnp.dot(x_ref[...], w_ref[...], preferred_element_type=jnp.float32)
    hp = h * _dis_from(deg_ref)[:, None]
    h0_ref[...] = hp[:, :DH]
    h1_ref[...] = hp[:, DH:]


def _mm_call(x_pad, W, deg2):
    return pl.pallas_call(
        _mm_body,
        grid=(N_PAD // BLK,),
        in_specs=[
            pl.BlockSpec((BLK, D), lambda i: (i, 0)),
            pl.BlockSpec((D, D), lambda i: (0, 0)),
            pl.BlockSpec((2, BLK), lambda i: (0, i)),
        ],
        out_specs=[
            pl.BlockSpec((BLK, DH), lambda i: (i, 0)),
            pl.BlockSpec((BLK, DH), lambda i: (i, 0)),
        ],
        out_shape=[
            jax.ShapeDtypeStruct((N_PAD, DH), jnp.float32),
            jax.ShapeDtypeStruct((N_PAD, DH), jnp.float32),
        ],
    )(x_pad, W, deg2)


def kernel(x, edge_index, W, b):
    src = edge_index[0]
    dst = edge_index[1]
    pad_n = E_PAD - N_EDGES
    pad_idx = N_NODES + (jnp.arange(pad_n, dtype=jnp.int32) % (N_PAD - N_NODES))
    srcp = jnp.concatenate([src, pad_idx])
    dstp = jnp.concatenate([dst, pad_idx])
    dst_deg = dstp.reshape(32, DEG_CHUNKS, CHUNK)
    src_agg = srcp.reshape(NT, AGG_CHUNKS, CHUNK)
    dst_agg = dstp.reshape(NT, AGG_CHUNKS, CHUNK)
    x_pad = jnp.pad(x, ((0, N_PAD - N_NODES), (0, 0)))
    zeros_n = jnp.zeros((N_PAD,), jnp.float32)
    ones_c = jnp.ones((CHUNK,), jnp.float32)

    deg_a, deg_b = _deg_call(dst_deg, zeros_n, ones_c)
    deg2 = jnp.stack([deg_a, deg_b])
    h0, h1 = _mm_call(x_pad, W, deg2)
    return _agg_call(h0, h1, src_agg, dst_agg, deg_a, deg_b, b)


# speedup vs baseline: 42.3943x; 1.0122x over previous
import jax
import jax.numpy as jnp
from jax import lax
from jax.experimental import pallas as pl
from jax.experimental.pallas import tpu as pltpu
from jax.experimental.pallas import tpu_sc as plsc

N_NODES = 10000
N_EDGES = 320000
D = 128
DH = 64
N_PAD = 10112
CHUNK = 128
NT = 16
ROWS_PER_TILE = N_PAD // NT

E_PAD = 327680
DEG_CHUNKS = 80
AGG_CHUNKS = 160
NBUF = 5

_MESH = plsc.VectorSubcoreMesh(core_axis_name="c", subcore_axis_name="s")


def _tile_1d_ranges(s, fn):
    @pl.when(s < NT - 1)
    def _():
        fn(s * 640, 640)

    @pl.when(s == NT - 1)
    def _():
        fn((NT - 1) * 640, N_PAD - (NT - 1) * 640)


def _deg_body(dst_hbm, zeros_hbm, ones_hbm, deg_out_a, deg_out_b,
              dst_v, ones_v, deg_sh, deg_sem):
    c = lax.axis_index("c")
    s = lax.axis_index("s")
    w = c * NT + s
    _tile_1d_ranges(s, lambda lo, n: pltpu.sync_copy(
        zeros_hbm.at[pl.ds(lo, n)], deg_sh.at[pl.ds(lo, n)]))
    pltpu.sync_copy(ones_hbm, ones_v)
    pltpu.sync_copy(dst_hbm.at[w], dst_v)
    plsc.subcore_barrier()

    def body(j, carry):
        pltpu.async_copy(ones_v, deg_sh.at[dst_v.at[j]], deg_sem, add=True)
        return carry

    lax.fori_loop(0, DEG_CHUNKS, body, 0)

    def drain(j, carry):
        pltpu.make_async_copy(ones_v, deg_sh.at[dst_v.at[j]], deg_sem).wait()
        return carry

    lax.fori_loop(0, DEG_CHUNKS, drain, 0)
    plsc.subcore_barrier()

    @pl.when(c == 0)
    def _():
        _tile_1d_ranges(s, lambda lo, n: pltpu.sync_copy(
            deg_sh.at[pl.ds(lo, n)], deg_out_a.at[pl.ds(lo, n)]))

    @pl.when(c == 1)
    def _():
        _tile_1d_ranges(s, lambda lo, n: pltpu.sync_copy(
            deg_sh.at[pl.ds(lo, n)], deg_out_b.at[pl.ds(lo, n)]))


_deg_call = pl.kernel(
    _deg_body,
    out_type=(jax.ShapeDtypeStruct((N_PAD,), jnp.float32),
              jax.ShapeDtypeStruct((N_PAD,), jnp.float32)),
    mesh=_MESH,
    scratch_types=[
        pltpu.VMEM((DEG_CHUNKS, CHUNK), jnp.int32),
        pltpu.VMEM((CHUNK,), jnp.float32),
        pltpu.VMEM_SHARED((N_PAD,), jnp.float32),
        pltpu.SemaphoreType.DMA,
    ],
)


def _agg_body(h0_hbm, h1_hbm, src_hbm, dst_hbm, deg_a_hbm, deg_b_hbm, b_hbm,
              out_hbm, src_v, dst_v, rows0, rows1, rows2, rows3, rows4,
              deg_va, deg_vb, dis_v, b_v, agg_sh, *sems):
    rows = (rows0, rows1, rows2, rows3, rows4)
    gsem = sems[:NBUF]
    ssem = sems[NBUF:]
    c = lax.axis_index("c")
    s = lax.axis_index("s")

    def zbody(i, carry):
        for j in range(DH // 16):
            rows0[i, pl.ds(j * 16, 16)] = jnp.zeros((16,), jnp.float32)
        return carry

    lax.fori_loop(0, CHUNK, zbody, 0)
    r0 = s * ROWS_PER_TILE
    nfull = ROWS_PER_TILE // CHUNK
    for k in range(nfull):
        pltpu.sync_copy(rows0, agg_sh.at[pl.ds(r0 + k * CHUNK, CHUNK)])
    rem = ROWS_PER_TILE % CHUNK
    if rem:
        pltpu.sync_copy(rows0.at[pl.ds(0, rem)],
                        agg_sh.at[pl.ds(r0 + nfull * CHUNK, rem)])
    pltpu.sync_copy(src_hbm.at[s], src_v)
    pltpu.sync_copy(dst_hbm.at[s], dst_v)
    plsc.subcore_barrier()

    def edge_loop(h_hbm):
        def start_g(t, b):
            pltpu.async_copy(h_hbm.at[src_v.at[t]], rows[b], gsem[b])

        def wait_g(t, b):
            pltpu.make_async_copy(h_hbm.at[src_v.at[t]], rows[b],
                                  gsem[b]).wait()

        def start_s(t, b):
            pltpu.async_copy(rows[b], agg_sh.at[dst_v.at[t]], ssem[b],
                             add=True)

        def wait_s(t, b):
            pltpu.make_async_copy(rows[b], agg_sh.at[dst_v.at[t]],
                                  ssem[b]).wait()

        def slot(t, b, with_ws, with_sg):
            wait_g(t, b)
            start_s(t, b)
            if with_ws:
                wait_s(t - 2, (b + 3) % NBUF)
            if with_sg:
                start_g(t + 3, (b + 3) % NBUF)

        start_g(0, 0)
        start_g(1, 1)
        start_g(2, 2)
        slot(0, 0, False, True)
        slot(1, 1, False, True)
        slot(2, 2, True, True)
        slot(3, 3, True, True)
        slot(4, 4, True, True)

        def round_body(g, carry):
            t0 = g * NBUF
            for b in range(NBUF):
                slot(t0 + b, b, True, True)
            return carry

        lax.fori_loop(1, AGG_CHUNKS // NBUF - 1, round_body, 0)
        t0 = AGG_CHUNKS - NBUF
        slot(t0 + 0, 0, True, True)
        slot(t0 + 1, 1, True, True)
        slot(t0 + 2, 2, True, False)
        slot(t0 + 3, 3, True, False)
        slot(t0 + 4, 4, True, False)
        wait_s(AGG_CHUNKS - 2, (AGG_CHUNKS - 2) % NBUF)
        wait_s(AGG_CHUNKS - 1, (AGG_CHUNKS - 1) % NBUF)

    @pl.when(c == 0)
    def _():
        edge_loop(h0_hbm)

    @pl.when(c == 1)
    def _():
        edge_loop(h1_hbm)

    plsc.subcore_barrier()

    off = jnp.where(s == NT - 1, 8, 0)
    lo = r0 - off
    pltpu.sync_copy(deg_a_hbm.at[pl.ds(lo, 640)], deg_va)
    pltpu.sync_copy(deg_b_hbm.at[pl.ds(lo, 640)], deg_vb)
    pltpu.sync_copy(b_hbm, b_v)

    def newton(k, carry):
        da = deg_va[pl.ds(k * 16, 16)] + deg_vb[pl.ds(k * 16, 16)]
        xi = plsc.bitcast(da, jnp.int32)
        yi = jnp.int32(0x5F3759DF) - lax.shift_right_logical(xi, 1)
        y = plsc.bitcast(yi, jnp.float32)
        for _ in range(4):
            y = y * (1.5 - 0.5 * da * y * y)
        dis_v[pl.ds(k * 16, 16)] = jnp.where(da > 0, y, 0.0)
        return carry

    lax.fori_loop(0, 640 // 16, newton, 0)
    bvs = [b_v[pl.ds(c * DH + j * 16, 16)] for j in range(DH // 16)]

    FB = 79

    def wr(start, rows_n):
        @pl.when(c == 0)
        def _():
            pltpu.sync_copy(rows0.at[pl.ds(0, rows_n)],
                            out_hbm.at[pl.ds(start, rows_n), pl.ds(0, DH)])

        @pl.when(c == 1)
        def _():
            pltpu.sync_copy(rows0.at[pl.ds(0, rows_n)],
                            out_hbm.at[pl.ds(start, rows_n), pl.ds(DH, DH)])

    for blk in range(ROWS_PER_TILE // FB):
        row_off = blk * FB
        start = r0 + row_off
        pltpu.sync_copy(agg_sh.at[pl.ds(start, FB)], rows0.at[pl.ds(0, FB)])

        def rowfix(r, carry):
            db = plsc.load_gather(
                dis_v, [jnp.full((16,), off + row_off, jnp.int32) + r])
            for j in range(DH // 16):
                v = rows0[r, pl.ds(j * 16, 16)]
                rows0[r, pl.ds(j * 16, 16)] = jnp.maximum(v * db + bvs[j], 0.0)
            return carry

        lax.fori_loop(0, FB, rowfix, 0)

        @pl.when(start + FB <= N_NODES)
        def _():
            wr(start, FB)

        @pl.when(jnp.logical_and(start < N_NODES, start + FB > N_NODES))
        def _():
            wr(start, N_NODES - (15 * ROWS_PER_TILE + 6 * FB))


_agg_call = pl.kernel(
    _agg_body,
    out_type=jax.ShapeDtypeStruct((N_NODES, D), jnp.float32),
    mesh=_MESH,
    scratch_types=[
        pltpu.VMEM((AGG_CHUNKS, CHUNK), jnp.int32),
        pltpu.VMEM((AGG_CHUNKS, CHUNK), jnp.int32),
        pltpu.VMEM((CHUNK, DH), jnp.float32),
        pltpu.VMEM((CHUNK, DH), jnp.float32),
        pltpu.VMEM((CHUNK, DH), jnp.float32),
        pltpu.VMEM((CHUNK, DH), jnp.float32),
        pltpu.VMEM((CHUNK, DH), jnp.float32),
        pltpu.VMEM((640,), jnp.float32),
        pltpu.VMEM((640,), jnp.float32),
        pltpu.VMEM((640,), jnp.float32),
        pltpu.VMEM((D,), jnp.float32),
        pltpu.VMEM_SHARED((N_PAD, DH), jnp.float32),
    ] + [pltpu.SemaphoreType.DMA] * (2 * NBUF),
    compiler_params=pltpu.CompilerParams(use_tc_tiling_on_sc=False,
                                         needs_layout_passes=False),
)

BLK = N_PAD


def _mm_body(x_ref, w_ref, deg_a_ref, deg_b_ref, h0_ref, h1_ref):
    h = jnp.dot(x_ref[...], w_ref[...], preferred_element_type=jnp.float32)
    deg = deg_a_ref[pl.ds(0, N_NODES)] + deg_b_ref[pl.ds(0, N_NODES)]
    dis = jnp.where(deg > 0, lax.rsqrt(jnp.maximum(deg, 1e-12)), 0.0)
    hp = h * dis[:, None]
    h0_ref[pl.ds(0, N_NODES), :] = hp[:, :DH]
    h1_ref[pl.ds(0, N_NODES), :] = hp[:, DH:]
    pad = jnp.zeros((N_PAD - N_NODES, DH), jnp.float32)
    h0_ref[pl.ds(N_NODES, N_PAD - N_NODES), :] = pad
    h1_ref[pl.ds(N_NODES, N_PAD - N_NODES), :] = pad


def _mm_call(x, W, deg_a, deg_b):
    return pl.pallas_call(
        _mm_body,
        grid=(1,),
        in_specs=[
            pl.BlockSpec((N_NODES, D), lambda i: (0, 0)),
            pl.BlockSpec((D, D), lambda i: (0, 0)),
            pl.BlockSpec((N_PAD,), lambda i: (0,)),
            pl.BlockSpec((N_PAD,), lambda i: (0,)),
        ],
        out_specs=[
            pl.BlockSpec((N_PAD, DH), lambda i: (0, 0)),
            pl.BlockSpec((N_PAD, DH), lambda i: (0, 0)),
        ],
        out_shape=[
            jax.ShapeDtypeStruct((N_PAD, DH), jnp.float32),
            jax.ShapeDtypeStruct((N_PAD, DH), jnp.float32),
        ],
    )(x, W, deg_a, deg_b)


def kernel(x, edge_index, W, b):
    src = edge_index[0]
    dst = edge_index[1]
    pad_n = E_PAD - N_EDGES
    pad_idx = N_NODES + (jnp.arange(pad_n, dtype=jnp.int32) % (N_PAD - N_NODES))
    srcp = jnp.concatenate([src, pad_idx])
    dstp = jnp.concatenate([dst, pad_idx])
    dst_deg = dstp.reshape(32, DEG_CHUNKS, CHUNK)
    src_agg = srcp.reshape(NT, AGG_CHUNKS, CHUNK)
    dst_agg = dstp.reshape(NT, AGG_CHUNKS, CHUNK)
    zeros_n = jnp.zeros((N_PAD,), jnp.float32)
    ones_c = jnp.ones((CHUNK,), jnp.float32)

    deg_a, deg_b = _deg_call(dst_deg, zeros_n, ones_c)
    h0, h1 = _mm_call(x, W, deg_a, deg_b)
    return _agg_call(h0, h1, src_agg, dst_agg, deg_a, deg_b, b)
